# Initial kernel scaffold; baseline (speedup 1.0000x reference)
import jax
import jax.numpy as jnp
from jax.experimental import pallas as pl


def kernel(edge_index, edge_type, edge_norm, subj, rel, init_embed, init_rel, w_loop, w_in, w_out, w_rel, loop_rel, conv1_bias, bn_c1_g, bn_c1_b, ent_bias, bn0_g, bn0_b, conv2d_w, conv2d_b, bn1_g, bn1_b, fc_w, fc_b, bn2_g, bn2_b):
    raise NotImplementedError("write your pallas kernel here")


# speedup vs baseline: 1.5778x; 1.5778x over previous
import functools

import jax
import jax.numpy as jnp
import numpy as np
from jax import lax
from jax.experimental import pallas as pl
from jax.experimental.pallas import tpu as pltpu
from jax.experimental.pallas import tpu_sc as plsc

NUM_ENT = 10000
D = 128
E = 320000
HALF = E // 2
B = 1024
NF = 96
KER = 7
NPOS = 100
FLAT = NF * NPOS

NC, NS = 2, 16
ET = HALF // NS
CH = 80
NCH = ET // CH
WT = 10
RPT = NUM_ENT // WT
ZR = 200

BB = 128
NBB = B // BB
SB = 128
NSB = B // SB

_f32 = jnp.float32


def _edge_aggregate(src_ids, dst_ids, edge_type, edge_norm, init_embed,
                    init_rel):
    mesh = plsc.VectorSubcoreMesh(core_axis_name="c", subcore_axis_name="s")

    @functools.partial(
        pl.kernel,
        mesh=mesh,
        out_type=jax.ShapeDtypeStruct((2, NUM_ENT, D), _f32),
        scratch_types=[
            pltpu.VMEM((CH,), jnp.int32),
            pltpu.VMEM((CH,), jnp.int32),
            pltpu.VMEM((CH,), jnp.int32),
            pltpu.VMEM((CH,), _f32),
            pltpu.VMEM((CH, D), _f32),
            pltpu.VMEM((CH, D), _f32),
            pltpu.VMEM((ZR, D), _f32),
            pltpu.VMEM_SHARED((NUM_ENT, D), _f32),
            pltpu.SemaphoreType.DMA,
            pltpu.SemaphoreType.DMA,
        ],
    )
    def k(esrc, edst, et, en, emb, rel, out, sidx, didx, tidx, nrm, srows,
          rrows, zbuf, acc, sem1, sem2):
        c = lax.axis_index("c")
        s = lax.axis_index("s")

        def zrow(i, carry):
            for j in range(D // 16):
                zbuf[i, pl.ds(j * 16, 16)] = jnp.zeros((16,), _f32)
            return carry

        lax.fori_loop(0, ZR, zrow, 0)

        @pl.when(s < WT)
        def _():
            for q in range(RPT // ZR):
                pltpu.sync_copy(zbuf, acc.at[pl.ds(s * RPT + q * ZR, ZR)])

        plsc.subcore_barrier()

        base0 = c * HALF + s * ET

        def chunk(t, carry):
            base = base0 + t * CH
            pltpu.sync_copy(esrc.at[pl.ds(base, CH)], sidx)
            pltpu.sync_copy(edst.at[pl.ds(base, CH)], didx)
            pltpu.sync_copy(et.at[pl.ds(base, CH)], tidx)
            pltpu.sync_copy(en.at[pl.ds(base, CH)], nrm)
            cp1 = pltpu.async_copy(emb.at[sidx], srows, sem1)
            cp2 = pltpu.async_copy(rel.at[tidx], rrows, sem2)
            cp1.wait()
            cp2.wait()

            def gbody(g, gcarry):
                nvec = nrm[pl.ds(g * 16, 16)]
                for i in range(16):
                    nv = lax.gather(
                        nvec, jnp.full((16, 1), i, jnp.int32),
                        lax.GatherDimensionNumbers(
                            offset_dims=(), collapsed_slice_dims=(0,),
                            start_index_map=(0,)),
                        (1,), mode=lax.GatherScatterMode.PROMISE_IN_BOUNDS)
                    e = g * 16 + i
                    for j in range(D // 16):
                        sl = pl.ds(j * 16, 16)
                        srows[e, sl] = srows[e, sl] * rrows[e, sl] * nv
                return gcarry

            lax.fori_loop(0, CH // 16, gbody, 0)
            pltpu.sync_copy(srows, acc.at[didx], add=True)
            return carry

        lax.fori_loop(0, NCH, chunk, 0)
        plsc.subcore_barrier()

        @pl.when(s < WT)
        def _():
            pltpu.sync_copy(acc.at[pl.ds(s * RPT, RPT)],
                            out.at[c, pl.ds(s * RPT, RPT)])

    return k(src_ids, dst_ids, edge_type, edge_norm, init_embed, init_rel)


def _gather_rows(x_pre, init_rel, subj, relidx):
    mesh = plsc.VectorSubcoreMesh(core_axis_name="c", subcore_axis_name="s")
    BW = B // (NC * NS)

    @functools.partial(
        pl.kernel,
        mesh=mesh,
        out_type=(jax.ShapeDtypeStruct((B, D), _f32),
                  jax.ShapeDtypeStruct((B, D), _f32)),
        scratch_types=[
            pltpu.VMEM((BW,), jnp.int32),
            pltpu.VMEM((BW,), jnp.int32),
            pltpu.VMEM((BW, D), _f32),
            pltpu.VMEM((BW, D), _f32),
            pltpu.SemaphoreType.DMA,
        ],
    )
    def k(xp, ir, sj, rl, osub, oir, iv1, iv2, r1, r2, sem):
        wid = lax.axis_index("s") * NC + lax.axis_index("c")
        bs = wid * BW
        pltpu.sync_copy(sj.at[pl.ds(bs, BW)], iv1)
        pltpu.sync_copy(rl.at[pl.ds(bs, BW)], iv2)
        cp1 = pltpu.async_copy(xp.at[iv1], r1, sem)
        cp2 = pltpu.async_copy(ir.at[iv2], r2, sem)
        cp1.wait()
        cp2.wait()
        pltpu.sync_copy(r1, osub.at[pl.ds(bs, BW)])
        pltpu.sync_copy(r2, oir.at[pl.ds(bs, BW)])

    return k(x_pre, init_rel, subj, relidx)


def _node_update(acc0, acc1, init_embed, w_in, w_out, w_loop, loop_rel,
                 conv1_bias):
    RA = 2000
    NBA = NUM_ENT // RA

    def body(a0, a1, emb, wi, wo, wl, lr, cb, xout, stat, accsc):
        i = pl.program_id(0)
        z = (jnp.dot(a0[...], wi[...], preferred_element_type=_f32)
             + jnp.dot(a1[...], wo[...], preferred_element_type=_f32)
             + jnp.dot(emb[...] * lr[...], wl[...],
                       preferred_element_type=_f32))
        z = z * (1.0 / 3.0) + cb[...]
        xout[...] = z

        @pl.when(i == 0)
        def _():
            accsc[...] = jnp.zeros_like(accsc)

        accsc[0:1, :] += jnp.sum(z, axis=0, keepdims=True)
        accsc[1:2, :] += jnp.sum(z * z, axis=0, keepdims=True)
        stat[...] = accsc[...]

    full = lambda shape: pl.BlockSpec(shape, lambda i: (0,) * len(shape))
    return pl.pallas_call(
        body,
        grid=(NBA,),
        in_specs=[
            pl.BlockSpec((RA, D), lambda i: (i, 0)),
            pl.BlockSpec((RA, D), lambda i: (i, 0)),
            pl.BlockSpec((RA, D), lambda i: (i, 0)),
            full((D, D)), full((D, D)), full((D, D)),
            full((1, D)), full((1, D)),
        ],
        out_specs=[pl.BlockSpec((RA, D), lambda i: (i, 0)), full((2, D))],
        out_shape=[jax.ShapeDtypeStruct((NUM_ENT, D), _f32),
                   jax.ShapeDtypeStruct((2, D), _f32)],
        scratch_shapes=[pltpu.VMEM((2, D), _f32)],
    )(acc0, acc1, init_embed, w_in, w_out, w_loop, loop_rel, conv1_bias)


def _conve_front(subf, irf, subb, irb, wrel, axr, bxr, g0, b0, g2mat, b96):
    sub_full = jnp.tanh(subf * axr + bxr)
    rel_full = jnp.dot(irf, wrel, preferred_element_type=_f32)
    n0 = 2.0 * B * D
    s0 = jnp.sum(sub_full) + jnp.sum(rel_full)
    ss0 = jnp.sum(sub_full * sub_full) + jnp.sum(rel_full * rel_full)
    m0 = s0 / n0
    v0 = ss0 / n0 - m0 * m0
    sc0 = g0[0, 0] * lax.rsqrt(v0 + 1e-5)
    sh0 = b0[0, 0] - m0 * sc0
    sub_blk = jnp.tanh(subb * axr + bxr) * sc0 + sh0
    rel_blk = jnp.dot(irb, wrel, preferred_element_type=_f32) * sc0 + sh0
    img = jnp.concatenate([sub_blk, rel_blk], axis=1)
    return jnp.dot(img, g2mat, preferred_element_type=_f32) + b96


def _conve_stats(subp, irp, w_rel, axr, bxr, g0, b0, g2mat, b96):
    def body(subf, irf, subb, irb, wrel, ax, bx, gg0, bb0, g2m, bb96,
             stat, accsc):
        i = pl.program_id(0)
        conv = _conve_front(subf[...], irf[...], subb[...], irb[...],
                            wrel[...], ax[...], bx[...], gg0[...], bb0[...],
                            g2m[...], bb96[...])

        @pl.when(i == 0)
        def _():
            accsc[...] = jnp.zeros_like(accsc)

        accsc[0:1, :] += jnp.sum(conv, axis=0, keepdims=True)
        accsc[1:2, :] += jnp.sum(conv * conv, axis=0, keepdims=True)
        stat[...] = accsc[...]

    full = lambda shape: pl.BlockSpec(shape, lambda i: (0,) * len(shape))
    return pl.pallas_call(
        body,
        grid=(NBB,),
        in_specs=[
            full((B, D)), full((B, D)),
            pl.BlockSpec((BB, D), lambda i: (i, 0)),
            pl.BlockSpec((BB, D), lambda i: (i, 0)),
            full((D, D)), full((1, D)), full((1, D)),
            full((1, 1)), full((1, 1)),
            full((2 * D, FLAT)), full((1, FLAT)),
        ],
        out_specs=full((2, FLAT)),
        out_shape=jax.ShapeDtypeStruct((2, FLAT), _f32),
        scratch_shapes=[pltpu.VMEM((2, FLAT), _f32)],
    )(subp, irp, subp, irp, w_rel, axr, bxr, g0, b0, g2mat, b96)


def _conve_apply(subp, irp, w_rel, axr, bxr, g0, b0, g2mat, b96,
                 alpha, beta, fcT, fcb):
    def body(subf, irf, subb, irb, wrel, ax, bx, gg0, bb0, g2m, bb96,
             al, be, fw, fb, hout):
        conv = _conve_front(subf[...], irf[...], subb[...], irb[...],
                            wrel[...], ax[...], bx[...], gg0[...], bb0[...],
                            g2m[...], bb96[...])
        y = jnp.maximum(conv * al[...] + be[...], 0.0)
        hout[...] = jnp.dot(y, fw[...], preferred_element_type=_f32) + fb[...]

    full = lambda shape: pl.BlockSpec(shape, lambda i: (0,) * len(shape))
    return pl.pallas_call(
        body,
        grid=(NBB,),
        in_specs=[
            full((B, D)), full((B, D)),
            pl.BlockSpec((BB, D), lambda i: (i, 0)),
            pl.BlockSpec((BB, D), lambda i: (i, 0)),
            full((D, D)), full((1, D)), full((1, D)),
            full((1, 1)), full((1, 1)),
            full((2 * D, FLAT)), full((1, FLAT)),
            full((1, FLAT)), full((1, FLAT)),
            full((FLAT, D)), full((1, D)),
        ],
        out_specs=pl.BlockSpec((BB, D), lambda i: (i, 0)),
        out_shape=jax.ShapeDtypeStruct((B, D), _f32),
    )(subp, irp, subp, irp, w_rel, axr, bxr, g0, b0, g2mat, b96,
      alpha, beta, fcT, fcb)


def _score(h_pre, x_pre, axr, bxr, g2r, b2r, ent_bias):
    def body(hf, hb_ref, xp, ax, bx, g2, b2, eb, score):
        hp = hf[...]
        m = jnp.mean(hp, axis=0, keepdims=True)
        v = jnp.mean(hp * hp, axis=0, keepdims=True) - m * m
        sc2 = lax.rsqrt(v + 1e-5) * g2[...]
        hb = jnp.maximum((hb_ref[...] - m) * sc2 + b2[...], 0.0)
        xt = jnp.tanh(xp[...] * ax[...] + bx[...])
        sc = lax.dot_general(hb, xt, (((1,), (1,)), ((), ())),
                             preferred_element_type=_f32)
        score[...] = 1.0 / (1.0 + jnp.exp(-(sc + eb[...])))

    full = lambda shape: pl.BlockSpec(shape, lambda i: (0,) * len(shape))
    return pl.pallas_call(
        body,
        grid=(NSB,),
        in_specs=[
            full((B, D)),
            pl.BlockSpec((SB, D), lambda i: (i, 0)),
            full((NUM_ENT, D)),
            full((1, D)), full((1, D)), full((1, D)), full((1, D)),
            full((1, NUM_ENT)),
        ],
        out_specs=pl.BlockSpec((SB, NUM_ENT), lambda i: (i, 0)),
        out_shape=jax.ShapeDtypeStruct((B, NUM_ENT), _f32),
    )(h_pre, h_pre, x_pre, axr, bxr, g2r, b2r, ent_bias)


def _build_toeplitz(conv2d_w):
    w = conv2d_w.reshape(NF, KER, KER)
    p = np.arange(NPOS)
    t = np.arange(KER * KER)
    P, T = np.meshgrid(p, t, indexing="ij")
    yv, xv = P // 10, P % 10
    kyv, kxv = T // KER, T % KER
    rows = (yv + kyv) * 16 + (xv + kxv)
    f = np.arange(NF)
    rows_full = np.broadcast_to(rows[None], (NF, NPOS, KER * KER)).ravel()
    cols_full = np.broadcast_to((f[:, None, None] * NPOS + P[None]),
                                (NF, NPOS, KER * KER)).ravel()
    vals = w[:, jnp.asarray(kyv), jnp.asarray(kxv)]
    return jnp.zeros((2 * D, FLAT), _f32).at[
        jnp.asarray(rows_full), jnp.asarray(cols_full)].add(vals.ravel())


def kernel(edge_index, edge_type, edge_norm, subj, rel, init_embed, init_rel,
           w_loop, w_in, w_out, w_rel, loop_rel, conv1_bias, bn_c1_g, bn_c1_b,
           ent_bias, bn0_g, bn0_b, conv2d_w, conv2d_b, bn1_g, bn1_b, fc_w,
           fc_b, bn2_g, bn2_b):
    acc = _edge_aggregate(edge_index[0], edge_index[1], edge_type, edge_norm,
                          init_embed, init_rel)

    x_pre, xstat = _node_update(acc[0], acc[1], init_embed, w_in, w_out,
                                w_loop, loop_rel.reshape(1, D),
                                conv1_bias.reshape(1, D))
    mean_x = xstat[0] / NUM_ENT
    var_x = xstat[1] / NUM_ENT - mean_x * mean_x
    ax = bn_c1_g * lax.rsqrt(var_x + 1e-5)
    bx = bn_c1_b - mean_x * ax
    axr, bxr = ax.reshape(1, D), bx.reshape(1, D)

    subp, irp = _gather_rows(x_pre, init_rel, subj, rel)

    g2mat = _build_toeplitz(conv2d_w)
    b96 = jnp.repeat(conv2d_b, NPOS).reshape(1, FLAT)
    g0 = bn0_g.reshape(1, 1)
    b0 = bn0_b.reshape(1, 1)

    cstat = _conve_stats(subp, irp, w_rel, axr, bxr, g0, b0, g2mat, b96)
    n1 = float(B * NPOS)
    sums = cstat[0].reshape(NF, NPOS).sum(axis=1)
    sumsq = cstat[1].reshape(NF, NPOS).sum(axis=1)
    mf = sums / n1
    vf = sumsq / n1 - mf * mf
    af = bn1_g * lax.rsqrt(vf + 1e-5)
    bf = bn1_b - mf * af
    alpha = jnp.repeat(af, NPOS).reshape(1, FLAT)
    beta = jnp.repeat(bf, NPOS).reshape(1, FLAT)

    h_pre = _conve_apply(subp, irp, w_rel, axr, bxr, g0, b0, g2mat, b96,
                         alpha, beta, fc_w.T, fc_b.reshape(1, D))

    return _score(h_pre, x_pre, axr, bxr, bn2_g.reshape(1, D),
                  bn2_b.reshape(1, D), ent_bias.reshape(1, NUM_ENT))


# speedup vs baseline: 3.2873x; 2.0835x over previous
import functools

import jax
import jax.numpy as jnp
import numpy as np
from jax import lax
from jax.experimental import pallas as pl
from jax.experimental.pallas import tpu as pltpu
from jax.experimental.pallas import tpu_sc as plsc

NUM_ENT = 10000
D = 128
E = 320000
HALF = E // 2
B = 1024
NF = 96
KER = 7
NPOS = 100
FLAT = NF * NPOS

NC, NS = 2, 16
ET = HALF // NS
CH = 80
NCH = ET // CH
WT = 10
RPT = NUM_ENT // WT
ZR = 200

BB = 128
NBB = B // BB
SB = 128
NSB = B // SB

_f32 = jnp.float32


def _edge_aggregate(src_ids, dst_ids, edge_type, edge_norm, init_embed,
                    init_rel):
    mesh = plsc.VectorSubcoreMesh(core_axis_name="c", subcore_axis_name="s")

    @functools.partial(
        pl.kernel,
        mesh=mesh,
        out_type=jax.ShapeDtypeStruct((2, NUM_ENT, D), _f32),
        scratch_types=[
            pltpu.VMEM((CH,), jnp.int32),
            pltpu.VMEM((CH,), jnp.int32),
            pltpu.VMEM((CH,), jnp.int32),
            pltpu.VMEM((CH,), _f32),
            pltpu.VMEM((CH, D), _f32),
            pltpu.VMEM((CH, D), _f32),
            pltpu.VMEM((ZR, D), _f32),
            pltpu.VMEM_SHARED((NUM_ENT, D), _f32),
            pltpu.SemaphoreType.DMA,
            pltpu.SemaphoreType.DMA,
        ],
    )
    def k(esrc, edst, et, en, emb, rel, out, sidx, didx, tidx, nrm, srows,
          rrows, zbuf, acc, sem1, sem2):
        c = lax.axis_index("c")
        s = lax.axis_index("s")

        def zrow(i, carry):
            for j in range(D // 16):
                zbuf[i, pl.ds(j * 16, 16)] = jnp.zeros((16,), _f32)
            return carry

        lax.fori_loop(0, ZR, zrow, 0)

        @pl.when(s < WT)
        def _():
            for q in range(RPT // ZR):
                pltpu.sync_copy(zbuf, acc.at[pl.ds(s * RPT + q * ZR, ZR)])

        plsc.subcore_barrier()

        base0 = c * HALF + s * ET

        def chunk(t, carry):
            base = base0 + t * CH
            pltpu.sync_copy(esrc.at[pl.ds(base, CH)], sidx)
            pltpu.sync_copy(edst.at[pl.ds(base, CH)], didx)
            pltpu.sync_copy(et.at[pl.ds(base, CH)], tidx)
            pltpu.sync_copy(en.at[pl.ds(base, CH)], nrm)
            cp1 = pltpu.async_copy(emb.at[sidx], srows, sem1)
            cp2 = pltpu.async_copy(rel.at[tidx], rrows, sem2)
            cp1.wait()
            cp2.wait()

            def gbody(g, gcarry):
                nvec = nrm[pl.ds(g * 16, 16)]
                for i in range(16):
                    nv = lax.gather(
                        nvec, jnp.full((16, 1), i, jnp.int32),
                        lax.GatherDimensionNumbers(
                            offset_dims=(), collapsed_slice_dims=(0,),
                            start_index_map=(0,)),
                        (1,), mode=lax.GatherScatterMode.PROMISE_IN_BOUNDS)
                    e = g * 16 + i
                    for j in range(D // 16):
                        sl = pl.ds(j * 16, 16)
                        srows[e, sl] = srows[e, sl] * rrows[e, sl] * nv
                return gcarry

            lax.fori_loop(0, CH // 16, gbody, 0)
            pltpu.sync_copy(srows, acc.at[didx], add=True)
            return carry

        lax.fori_loop(0, NCH, chunk, 0)
        plsc.subcore_barrier()

        @pl.when(s < WT)
        def _():
            pltpu.sync_copy(acc.at[pl.ds(s * RPT, RPT)],
                            out.at[c, pl.ds(s * RPT, RPT)])

    return k(src_ids, dst_ids, edge_type, edge_norm, init_embed, init_rel)


def _gather_rows(x_pre, init_rel, subj, relidx):
    mesh = plsc.VectorSubcoreMesh(core_axis_name="c", subcore_axis_name="s")
    BW = B // (NC * NS)

    @functools.partial(
        pl.kernel,
        mesh=mesh,
        out_type=(jax.ShapeDtypeStruct((B, D), _f32),
                  jax.ShapeDtypeStruct((B, D), _f32)),
        scratch_types=[
            pltpu.VMEM((BW,), jnp.int32),
            pltpu.VMEM((BW,), jnp.int32),
            pltpu.VMEM((BW, D), _f32),
            pltpu.VMEM((BW, D), _f32),
            pltpu.SemaphoreType.DMA,
        ],
    )
    def k(xp, ir, sj, rl, osub, oir, iv1, iv2, r1, r2, sem):
        wid = lax.axis_index("s") * NC + lax.axis_index("c")
        bs = wid * BW
        pltpu.sync_copy(sj.at[pl.ds(bs, BW)], iv1)
        pltpu.sync_copy(rl.at[pl.ds(bs, BW)], iv2)
        cp1 = pltpu.async_copy(xp.at[iv1], r1, sem)
        cp2 = pltpu.async_copy(ir.at[iv2], r2, sem)
        cp1.wait()
        cp2.wait()
        pltpu.sync_copy(r1, osub.at[pl.ds(bs, BW)])
        pltpu.sync_copy(r2, oir.at[pl.ds(bs, BW)])

    return k(x_pre, init_rel, subj, relidx)


def _node_update(acc0, acc1, init_embed, w_in, w_out, w_loop, loop_rel,
                 conv1_bias):
    RA = 2000
    NBA = NUM_ENT // RA

    def body(a0, a1, emb, wi, wo, wl, lr, cb, xout, stat, accsc):
        i = pl.program_id(0)
        z = (jnp.dot(a0[...], wi[...], preferred_element_type=_f32)
             + jnp.dot(a1[...], wo[...], preferred_element_type=_f32)
             + jnp.dot(emb[...] * lr[...], wl[...],
                       preferred_element_type=_f32))
        z = z * (1.0 / 3.0) + cb[...]
        xout[...] = z

        @pl.when(i == 0)
        def _():
            accsc[...] = jnp.zeros_like(accsc)

        accsc[0:1, :] += jnp.sum(z, axis=0, keepdims=True)
        accsc[1:2, :] += jnp.sum(z * z, axis=0, keepdims=True)
        stat[...] = accsc[...]

    full = lambda shape: pl.BlockSpec(shape, lambda i: (0,) * len(shape))
    return pl.pallas_call(
        body,
        grid=(NBA,),
        in_specs=[
            pl.BlockSpec((RA, D), lambda i: (i, 0)),
            pl.BlockSpec((RA, D), lambda i: (i, 0)),
            pl.BlockSpec((RA, D), lambda i: (i, 0)),
            full((D, D)), full((D, D)), full((D, D)),
            full((1, D)), full((1, D)),
        ],
        out_specs=[pl.BlockSpec((RA, D), lambda i: (i, 0)), full((2, D))],
        out_shape=[jax.ShapeDtypeStruct((NUM_ENT, D), _f32),
                   jax.ShapeDtypeStruct((2, D), _f32)],
        scratch_shapes=[pltpu.VMEM((2, D), _f32)],
    )(acc0, acc1, init_embed, w_in, w_out, w_loop, loop_rel, conv1_bias)


def _conve_front(subf, irf, subb, irb, wrel, axr, bxr, g0, b0, g2mat, b96):
    sub_full = jnp.tanh(subf * axr + bxr)
    rel_full = jnp.dot(irf, wrel, preferred_element_type=_f32)
    n0 = 2.0 * B * D
    s0 = jnp.sum(sub_full) + jnp.sum(rel_full)
    ss0 = jnp.sum(sub_full * sub_full) + jnp.sum(rel_full * rel_full)
    m0 = s0 / n0
    v0 = ss0 / n0 - m0 * m0
    sc0 = g0[0, 0] * lax.rsqrt(v0 + 1e-5)
    sh0 = b0[0, 0] - m0 * sc0
    sub_blk = jnp.tanh(subb * axr + bxr) * sc0 + sh0
    rel_blk = jnp.dot(irb, wrel, preferred_element_type=_f32) * sc0 + sh0
    img = jnp.concatenate([sub_blk, rel_blk], axis=1)
    return jnp.dot(img, g2mat, preferred_element_type=_f32) + b96


def _conve_stats(subp, irp, w_rel, axr, bxr, g0, b0, g2mat, b96):
    def body(subf, irf, subb, irb, wrel, ax, bx, gg0, bb0, g2m, bb96,
             stat, accsc):
        i = pl.program_id(0)
        conv = _conve_front(subf[...], irf[...], subb[...], irb[...],
                            wrel[...], ax[...], bx[...], gg0[...], bb0[...],
                            g2m[...], bb96[...])

        @pl.when(i == 0)
        def _():
            accsc[...] = jnp.zeros_like(accsc)

        accsc[0:1, :] += jnp.sum(conv, axis=0, keepdims=True)
        accsc[1:2, :] += jnp.sum(conv * conv, axis=0, keepdims=True)
        stat[...] = accsc[...]

    full = lambda shape: pl.BlockSpec(shape, lambda i: (0,) * len(shape))
    return pl.pallas_call(
        body,
        grid=(NBB,),
        in_specs=[
            full((B, D)), full((B, D)),
            pl.BlockSpec((BB, D), lambda i: (i, 0)),
            pl.BlockSpec((BB, D), lambda i: (i, 0)),
            full((D, D)), full((1, D)), full((1, D)),
            full((1, 1)), full((1, 1)),
            full((2 * D, FLAT)), full((1, FLAT)),
        ],
        out_specs=full((2, FLAT)),
        out_shape=jax.ShapeDtypeStruct((2, FLAT), _f32),
        scratch_shapes=[pltpu.VMEM((2, FLAT), _f32)],
    )(subp, irp, subp, irp, w_rel, axr, bxr, g0, b0, g2mat, b96)


def _conve_apply(subp, irp, w_rel, axr, bxr, g0, b0, g2mat, b96,
                 alpha, beta, fcT, fcb):
    def body(subf, irf, subb, irb, wrel, ax, bx, gg0, bb0, g2m, bb96,
             al, be, fw, fb, hout):
        conv = _conve_front(subf[...], irf[...], subb[...], irb[...],
                            wrel[...], ax[...], bx[...], gg0[...], bb0[...],
                            g2m[...], bb96[...])
        y = jnp.maximum(conv * al[...] + be[...], 0.0)
        hout[...] = jnp.dot(y, fw[...], preferred_element_type=_f32) + fb[...]

    full = lambda shape: pl.BlockSpec(shape, lambda i: (0,) * len(shape))
    return pl.pallas_call(
        body,
        grid=(NBB,),
        in_specs=[
            full((B, D)), full((B, D)),
            pl.BlockSpec((BB, D), lambda i: (i, 0)),
            pl.BlockSpec((BB, D), lambda i: (i, 0)),
            full((D, D)), full((1, D)), full((1, D)),
            full((1, 1)), full((1, 1)),
            full((2 * D, FLAT)), full((1, FLAT)),
            full((1, FLAT)), full((1, FLAT)),
            full((FLAT, D)), full((1, D)),
        ],
        out_specs=pl.BlockSpec((BB, D), lambda i: (i, 0)),
        out_shape=jax.ShapeDtypeStruct((B, D), _f32),
    )(subp, irp, subp, irp, w_rel, axr, bxr, g0, b0, g2mat, b96,
      alpha, beta, fcT, fcb)


def _score(h_pre, x_pre, axr, bxr, g2r, b2r, ent_bias):
    def body(hf, hb_ref, xp, ax, bx, g2, b2, eb, score):
        hp = hf[...]
        m = jnp.mean(hp, axis=0, keepdims=True)
        v = jnp.mean(hp * hp, axis=0, keepdims=True) - m * m
        sc2 = lax.rsqrt(v + 1e-5) * g2[...]
        hb = jnp.maximum((hb_ref[...] - m) * sc2 + b2[...], 0.0)
        xt = jnp.tanh(xp[...] * ax[...] + bx[...])
        sc = lax.dot_general(hb, xt, (((1,), (1,)), ((), ())),
                             preferred_element_type=_f32)
        score[...] = 1.0 / (1.0 + jnp.exp(-(sc + eb[...])))

    full = lambda shape: pl.BlockSpec(shape, lambda i: (0,) * len(shape))
    return pl.pallas_call(
        body,
        grid=(NSB,),
        in_specs=[
            full((B, D)),
            pl.BlockSpec((SB, D), lambda i: (i, 0)),
            full((NUM_ENT, D)),
            full((1, D)), full((1, D)), full((1, D)), full((1, D)),
            full((1, NUM_ENT)),
        ],
        out_specs=pl.BlockSpec((SB, NUM_ENT), lambda i: (i, 0)),
        out_shape=jax.ShapeDtypeStruct((B, NUM_ENT), _f32),
    )(h_pre, h_pre, x_pre, axr, bxr, g2r, b2r, ent_bias)


def _toeplitz_selector():
    a = np.zeros((2 * D * NPOS, KER * KER), np.float32)
    r = np.arange(2 * D)
    p = np.arange(NPOS)
    R, P = np.meshgrid(r, p, indexing="ij")
    ky = R // 16 - P // 10
    kx = R % 16 - P % 10
    ok = (ky >= 0) & (ky < KER) & (kx >= 0) & (kx < KER)
    t = np.where(ok, ky * KER + kx, 0)
    a[np.arange(2 * D * NPOS)[ok.ravel()], t.ravel()[ok.ravel()]] = 1.0
    return a


_A_SEL = _toeplitz_selector()
GR = 6400
NGR = (2 * D * NPOS) // GR


def _build_toeplitz(conv2d_w):
    w_flat = conv2d_w.reshape(NF, KER * KER)

    def body(a, w, o):
        o[...] = lax.dot_general(a[...], w[...], (((1,), (1,)), ((), ())),
                                 preferred_element_type=_f32)

    g = pl.pallas_call(
        body,
        grid=(NGR,),
        in_specs=[pl.BlockSpec((GR, KER * KER), lambda i: (i, 0)),
                  pl.BlockSpec((NF, KER * KER), lambda i: (0, 0))],
        out_specs=pl.BlockSpec((GR, NF), lambda i: (i, 0)),
        out_shape=jax.ShapeDtypeStruct((2 * D * NPOS, NF), _f32),
    )(jnp.asarray(_A_SEL), w_flat)
    return g.reshape(2 * D, FLAT)


def kernel(edge_index, edge_type, edge_norm, subj, rel, init_embed, init_rel,
           w_loop, w_in, w_out, w_rel, loop_rel, conv1_bias, bn_c1_g, bn_c1_b,
           ent_bias, bn0_g, bn0_b, conv2d_w, conv2d_b, bn1_g, bn1_b, fc_w,
           fc_b, bn2_g, bn2_b):
    acc = _edge_aggregate(edge_index[0], edge_index[1], edge_type, edge_norm,
                          init_embed, init_rel)

    x_pre, xstat = _node_update(acc[0], acc[1], init_embed, w_in, w_out,
                                w_loop, loop_rel.reshape(1, D),
                                conv1_bias.reshape(1, D))
    mean_x = xstat[0] / NUM_ENT
    var_x = xstat[1] / NUM_ENT - mean_x * mean_x
    ax = bn_c1_g * lax.rsqrt(var_x + 1e-5)
    bx = bn_c1_b - mean_x * ax
    axr, bxr = ax.reshape(1, D), bx.reshape(1, D)

    subp, irp = _gather_rows(x_pre, init_rel, subj, rel)

    g2mat = _build_toeplitz(conv2d_w)
    b96 = jnp.tile(conv2d_b, NPOS).reshape(1, FLAT)
    g0 = bn0_g.reshape(1, 1)
    b0 = bn0_b.reshape(1, 1)

    cstat = _conve_stats(subp, irp, w_rel, axr, bxr, g0, b0, g2mat, b96)
    n1 = float(B * NPOS)
    sums = cstat[0].reshape(NPOS, NF).sum(axis=0)
    sumsq = cstat[1].reshape(NPOS, NF).sum(axis=0)
    mf = sums / n1
    vf = sumsq / n1 - mf * mf
    af = bn1_g * lax.rsqrt(vf + 1e-5)
    bf = bn1_b - mf * af
    alpha = jnp.tile(af, NPOS).reshape(1, FLAT)
    beta = jnp.tile(bf, NPOS).reshape(1, FLAT)

    fcT = fc_w.reshape(D, NF, NPOS).transpose(2, 1, 0).reshape(FLAT, D)
    h_pre = _conve_apply(subp, irp, w_rel, axr, bxr, g0, b0, g2mat, b96,
                         alpha, beta, fcT, fc_b.reshape(1, D))

    return _score(h_pre, x_pre, axr, bxr, bn2_g.reshape(1, D),
                  bn2_b.reshape(1, D), ent_bias.reshape(1, NUM_ENT))


# speedup vs baseline: 3.8966x; 1.1853x over previous
import functools

import jax
import jax.numpy as jnp
import numpy as np
from jax import lax
from jax.experimental import pallas as pl
from jax.experimental.pallas import tpu as pltpu
from jax.experimental.pallas import tpu_sc as plsc

NUM_ENT = 10000
D = 128
E = 320000
HALF = E // 2
B = 1024
NF = 96
KER = 7
NPOS = 100
FLAT = NF * NPOS

NC, NS = 2, 16
ET = HALF // NS
CH = 80
NCH = ET // CH
WT = 10
RPT = NUM_ENT // WT
ZR = 8

BB = 128
NBB = B // BB
SB = 128
NSB = B // SB

_f32 = jnp.float32


def _edge_aggregate(src_ids, dst_ids, edge_type, edge_norm, init_embed,
                    init_rel):
    mesh = plsc.VectorSubcoreMesh(core_axis_name="c", subcore_axis_name="s")

    NREL = 200

    @functools.partial(
        pl.kernel,
        mesh=mesh,
        out_type=jax.ShapeDtypeStruct((2, NUM_ENT, D), _f32),
        scratch_types=[
            pltpu.VMEM((2, CH), jnp.int32),
            pltpu.VMEM((2, CH), jnp.int32),
            pltpu.VMEM((2, CH), jnp.int32),
            pltpu.VMEM((2, CH), _f32),
            pltpu.VMEM((2, CH, D), _f32),
            pltpu.VMEM((2, CH, D), _f32),
            pltpu.VMEM((ZR, D), _f32),
            pltpu.VMEM_SHARED((NUM_ENT, D), _f32),
            pltpu.SemaphoreType.DMA,
            pltpu.SemaphoreType.DMA,
        ],
    )
    def k(esrc, edst, et, en, emb, rel, out, sidx, didx, tidx, nrm, srows,
          rrows, zbuf, acc, sem1, sem2):
        c = lax.axis_index("c")
        s = lax.axis_index("s")

        def zrow(i, carry):
            for j in range(D // 16):
                zbuf[i, pl.ds(j * 16, 16)] = jnp.zeros((16,), _f32)
            return carry

        lax.fori_loop(0, ZR, zrow, 0)

        @pl.when(s < WT)
        def _():
            for q in range(RPT // ZR):
                pltpu.sync_copy(zbuf, acc.at[pl.ds(s * RPT + q * ZR, ZR)])

        plsc.subcore_barrier()

        base0 = c * HALF + s * ET

        def load_idx(t, b):
            base = base0 + t * CH
            pltpu.sync_copy(esrc.at[pl.ds(base, CH)], sidx.at[b])
            pltpu.sync_copy(edst.at[pl.ds(base, CH)], didx.at[b])
            pltpu.sync_copy(et.at[pl.ds(base, CH)], tidx.at[b])
            pltpu.sync_copy(en.at[pl.ds(base, CH)], nrm.at[b])

        def start_gathers(b, sem):
            pltpu.async_copy(emb.at[sidx.at[b]], srows.at[b], sem)
            pltpu.async_copy(rel.at[tidx.at[b]], rrows.at[b], sem)

        def drain(b, sem):
            pltpu.make_async_copy(emb.at[sidx.at[b]], srows.at[b], sem).wait()
            pltpu.make_async_copy(rel.at[tidx.at[b]], rrows.at[b],
                                  sem).wait()

        def compose_scatter(b):
            def gbody(g, gcarry):
                nvec = nrm[b, pl.ds(g * 16, 16)]
                for i in range(16):
                    nv = lax.gather(
                        nvec, jnp.full((16, 1), i, jnp.int32),
                        lax.GatherDimensionNumbers(
                            offset_dims=(), collapsed_slice_dims=(0,),
                            start_index_map=(0,)),
                        (1,), mode=lax.GatherScatterMode.PROMISE_IN_BOUNDS)
                    e = g * 16 + i
                    for j in range(D // 16):
                        sl = pl.ds(j * 16, 16)
                        srows[b, e, sl] = srows[b, e, sl] * rrows[b, e, sl] * nv
                return gcarry

            lax.fori_loop(0, CH // 16, gbody, 0)
            pltpu.sync_copy(srows.at[b], acc.at[didx.at[b]], add=True)

        load_idx(0, 0)
        start_gathers(0, sem1)

        def body(kk, carry):
            load_idx(2 * kk + 1, 1)
            start_gathers(1, sem2)
            drain(0, sem1)
            compose_scatter(0)
            load_idx(2 * kk + 2, 0)
            start_gathers(0, sem1)
            drain(1, sem2)
            compose_scatter(1)
            return carry

        lax.fori_loop(0, NCH // 2, body, 0)
        drain(0, sem1)
        compose_scatter(0)
        plsc.subcore_barrier()

        @pl.when(s < WT)
        def _():
            pltpu.sync_copy(acc.at[pl.ds(s * RPT, RPT)],
                            out.at[c, pl.ds(s * RPT, RPT)])

    return k(src_ids, dst_ids, edge_type, edge_norm, init_embed, init_rel)


def _gather_rows(x_pre, init_rel, subj, relidx):
    mesh = plsc.VectorSubcoreMesh(core_axis_name="c", subcore_axis_name="s")
    BW = B // (NC * NS)

    @functools.partial(
        pl.kernel,
        mesh=mesh,
        out_type=(jax.ShapeDtypeStruct((B, D), _f32),
                  jax.ShapeDtypeStruct((B, D), _f32)),
        scratch_types=[
            pltpu.VMEM((BW,), jnp.int32),
            pltpu.VMEM((BW,), jnp.int32),
            pltpu.VMEM((BW, D), _f32),
            pltpu.VMEM((BW, D), _f32),
            pltpu.SemaphoreType.DMA,
        ],
    )
    def k(xp, ir, sj, rl, osub, oir, iv1, iv2, r1, r2, sem):
        wid = lax.axis_index("s") * NC + lax.axis_index("c")
        bs = wid * BW
        pltpu.sync_copy(sj.at[pl.ds(bs, BW)], iv1)
        pltpu.sync_copy(rl.at[pl.ds(bs, BW)], iv2)
        cp1 = pltpu.async_copy(xp.at[iv1], r1, sem)
        cp2 = pltpu.async_copy(ir.at[iv2], r2, sem)
        cp1.wait()
        cp2.wait()
        pltpu.sync_copy(r1, osub.at[pl.ds(bs, BW)])
        pltpu.sync_copy(r2, oir.at[pl.ds(bs, BW)])

    return k(x_pre, init_rel, subj, relidx)


def _node_update(acc0, acc1, init_embed, w_in, w_out, w_loop, loop_rel,
                 conv1_bias):
    RA = 2000
    NBA = NUM_ENT // RA

    def body(a0, a1, emb, wi, wo, wl, lr, cb, xout, stat, accsc):
        i = pl.program_id(0)
        z = (jnp.dot(a0[...], wi[...], preferred_element_type=_f32)
             + jnp.dot(a1[...], wo[...], preferred_element_type=_f32)
             + jnp.dot(emb[...] * lr[...], wl[...],
                       preferred_element_type=_f32))
        z = z * (1.0 / 3.0) + cb[...]
        xout[...] = z

        @pl.when(i == 0)
        def _():
            accsc[...] = jnp.zeros_like(accsc)

        accsc[0:1, :] += jnp.sum(z, axis=0, keepdims=True)
        accsc[1:2, :] += jnp.sum(z * z, axis=0, keepdims=True)
        stat[...] = accsc[...]

    full = lambda shape: pl.BlockSpec(shape, lambda i: (0,) * len(shape))
    return pl.pallas_call(
        body,
        grid=(NBA,),
        in_specs=[
            pl.BlockSpec((RA, D), lambda i: (i, 0)),
            pl.BlockSpec((RA, D), lambda i: (i, 0)),
            pl.BlockSpec((RA, D), lambda i: (i, 0)),
            full((D, D)), full((D, D)), full((D, D)),
            full((1, D)), full((1, D)),
        ],
        out_specs=[pl.BlockSpec((RA, D), lambda i: (i, 0)), full((2, D))],
        out_shape=[jax.ShapeDtypeStruct((NUM_ENT, D), _f32),
                   jax.ShapeDtypeStruct((2, D), _f32)],
        scratch_shapes=[pltpu.VMEM((2, D), _f32)],
    )(acc0, acc1, init_embed, w_in, w_out, w_loop, loop_rel, conv1_bias)


def _conve_front(subf, irf, subb, irb, wrel, axr, bxr, g0, b0, g2mat, b96):
    sub_full = jnp.tanh(subf * axr + bxr)
    rel_full = jnp.dot(irf, wrel, preferred_element_type=_f32)
    n0 = 2.0 * B * D
    s0 = jnp.sum(sub_full) + jnp.sum(rel_full)
    ss0 = jnp.sum(sub_full * sub_full) + jnp.sum(rel_full * rel_full)
    m0 = s0 / n0
    v0 = ss0 / n0 - m0 * m0
    sc0 = g0[0, 0] * lax.rsqrt(v0 + 1e-5)
    sh0 = b0[0, 0] - m0 * sc0
    sub_blk = jnp.tanh(subb * axr + bxr) * sc0 + sh0
    rel_blk = jnp.dot(irb, wrel, preferred_element_type=_f32) * sc0 + sh0
    img = jnp.concatenate([sub_blk, rel_blk], axis=1)
    return jnp.dot(img, g2mat, preferred_element_type=_f32) + b96


def _conve_stats(subp, irp, w_rel, axr, bxr, g0, b0, g2mat, b96):
    def body(subf, irf, subb, irb, wrel, ax, bx, gg0, bb0, g2m, bb96,
             stat, accsc):
        i = pl.program_id(0)
        conv = _conve_front(subf[...], irf[...], subb[...], irb[...],
                            wrel[...], ax[...], bx[...], gg0[...], bb0[...],
                            g2m[...], bb96[...])

        @pl.when(i == 0)
        def _():
            accsc[...] = jnp.zeros_like(accsc)

        accsc[0:1, :] += jnp.sum(conv, axis=0, keepdims=True)
        accsc[1:2, :] += jnp.sum(conv * conv, axis=0, keepdims=True)
        stat[...] = accsc[...]

    full = lambda shape: pl.BlockSpec(shape, lambda i: (0,) * len(shape))
    return pl.pallas_call(
        body,
        grid=(NBB,),
        in_specs=[
            full((B, D)), full((B, D)),
            pl.BlockSpec((BB, D), lambda i: (i, 0)),
            pl.BlockSpec((BB, D), lambda i: (i, 0)),
            full((D, D)), full((1, D)), full((1, D)),
            full((1, 1)), full((1, 1)),
            full((2 * D, FLAT)), full((1, FLAT)),
        ],
        out_specs=full((2, FLAT)),
        out_shape=jax.ShapeDtypeStruct((2, FLAT), _f32),
        scratch_shapes=[pltpu.VMEM((2, FLAT), _f32)],
    )(subp, irp, subp, irp, w_rel, axr, bxr, g0, b0, g2mat, b96)


def _conve_apply(subp, irp, w_rel, axr, bxr, g0, b0, g2mat, b96,
                 alpha, beta, fcT, fcb):
    def body(subf, irf, subb, irb, wrel, ax, bx, gg0, bb0, g2m, bb96,
             al, be, fw, fb, hout):
        conv = _conve_front(subf[...], irf[...], subb[...], irb[...],
                            wrel[...], ax[...], bx[...], gg0[...], bb0[...],
                            g2m[...], bb96[...])
        y = jnp.maximum(conv * al[...] + be[...], 0.0)
        hout[...] = jnp.dot(y, fw[...], preferred_element_type=_f32) + fb[...]

    full = lambda shape: pl.BlockSpec(shape, lambda i: (0,) * len(shape))
    return pl.pallas_call(
        body,
        grid=(NBB,),
        in_specs=[
            full((B, D)), full((B, D)),
            pl.BlockSpec((BB, D), lambda i: (i, 0)),
            pl.BlockSpec((BB, D), lambda i: (i, 0)),
            full((D, D)), full((1, D)), full((1, D)),
            full((1, 1)), full((1, 1)),
            full((2 * D, FLAT)), full((1, FLAT)),
            full((1, FLAT)), full((1, FLAT)),
            full((FLAT, D)), full((1, D)),
        ],
        out_specs=pl.BlockSpec((BB, D), lambda i: (i, 0)),
        out_shape=jax.ShapeDtypeStruct((B, D), _f32),
    )(subp, irp, subp, irp, w_rel, axr, bxr, g0, b0, g2mat, b96,
      alpha, beta, fcT, fcb)


def _score(h_pre, x_pre, axr, bxr, g2r, b2r, ent_bias):
    def body(hf, hb_ref, xp, ax, bx, g2, b2, eb, score):
        hp = hf[...]
        m = jnp.mean(hp, axis=0, keepdims=True)
        v = jnp.mean(hp * hp, axis=0, keepdims=True) - m * m
        sc2 = lax.rsqrt(v + 1e-5) * g2[...]
        hb = jnp.maximum((hb_ref[...] - m) * sc2 + b2[...], 0.0)
        xt = jnp.tanh(xp[...] * ax[...] + bx[...])
        sc = lax.dot_general(hb, xt, (((1,), (1,)), ((), ())),
                             preferred_element_type=_f32)
        score[...] = 1.0 / (1.0 + jnp.exp(-(sc + eb[...])))

    full = lambda shape: pl.BlockSpec(shape, lambda i: (0,) * len(shape))
    return pl.pallas_call(
        body,
        grid=(NSB,),
        in_specs=[
            full((B, D)),
            pl.BlockSpec((SB, D), lambda i: (i, 0)),
            full((NUM_ENT, D)),
            full((1, D)), full((1, D)), full((1, D)), full((1, D)),
            full((1, NUM_ENT)),
        ],
        out_specs=pl.BlockSpec((SB, NUM_ENT), lambda i: (i, 0)),
        out_shape=jax.ShapeDtypeStruct((B, NUM_ENT), _f32),
    )(h_pre, h_pre, x_pre, axr, bxr, g2r, b2r, ent_bias)


def _toeplitz_selector():
    a = np.zeros((2 * D * NPOS, KER * KER), np.float32)
    r = np.arange(2 * D)
    p = np.arange(NPOS)
    R, P = np.meshgrid(r, p, indexing="ij")
    ky = R // 16 - P // 10
    kx = R % 16 - P % 10
    ok = (ky >= 0) & (ky < KER) & (kx >= 0) & (kx < KER)
    t = np.where(ok, ky * KER + kx, 0)
    a[np.arange(2 * D * NPOS)[ok.ravel()], t.ravel()[ok.ravel()]] = 1.0
    return a


_A_SEL = _toeplitz_selector()
GR = 6400
NGR = (2 * D * NPOS) // GR


def _build_toeplitz(conv2d_w):
    w_flat = conv2d_w.reshape(NF, KER * KER)

    def body(a, w, o):
        o[...] = lax.dot_general(a[...], w[...], (((1,), (1,)), ((), ())),
                                 preferred_element_type=_f32)

    g = pl.pallas_call(
        body,
        grid=(NGR,),
        in_specs=[pl.BlockSpec((GR, KER * KER), lambda i: (i, 0)),
                  pl.BlockSpec((NF, KER * KER), lambda i: (0, 0))],
        out_specs=pl.BlockSpec((GR, NF), lambda i: (i, 0)),
        out_shape=jax.ShapeDtypeStruct((2 * D * NPOS, NF), _f32),
    )(jnp.asarray(_A_SEL), w_flat)
    return g.reshape(2 * D, FLAT)


def kernel(edge_index, edge_type, edge_norm, subj, rel, init_embed, init_rel,
           w_loop, w_in, w_out, w_rel, loop_rel, conv1_bias, bn_c1_g, bn_c1_b,
           ent_bias, bn0_g, bn0_b, conv2d_w, conv2d_b, bn1_g, bn1_b, fc_w,
           fc_b, bn2_g, bn2_b):
    acc = _edge_aggregate(edge_index[0], edge_index[1], edge_type, edge_norm,
                          init_embed, init_rel)

    x_pre, xstat = _node_update(acc[0], acc[1], init_embed, w_in, w_out,
                                w_loop, loop_rel.reshape(1, D),
                                conv1_bias.reshape(1, D))
    mean_x = xstat[0] / NUM_ENT
    var_x = xstat[1] / NUM_ENT - mean_x * mean_x
    ax = bn_c1_g * lax.rsqrt(var_x + 1e-5)
    bx = bn_c1_b - mean_x * ax
    axr, bxr = ax.reshape(1, D), bx.reshape(1, D)

    subp, irp = _gather_rows(x_pre, init_rel, subj, rel)

    g2mat = _build_toeplitz(conv2d_w)
    b96 = jnp.tile(conv2d_b, NPOS).reshape(1, FLAT)
    g0 = bn0_g.reshape(1, 1)
    b0 = bn0_b.reshape(1, 1)

    cstat = _conve_stats(subp, irp, w_rel, axr, bxr, g0, b0, g2mat, b96)
    n1 = float(B * NPOS)
    sums = cstat[0].reshape(NPOS, NF).sum(axis=0)
    sumsq = cstat[1].reshape(NPOS, NF).sum(axis=0)
    mf = sums / n1
    vf = sumsq / n1 - mf * mf
    af = bn1_g * lax.rsqrt(vf + 1e-5)
    bf = bn1_b - mf * af
    alpha = jnp.tile(af, NPOS).reshape(1, FLAT)
    beta = jnp.tile(bf, NPOS).reshape(1, FLAT)

    fcT = fc_w.reshape(D, NF, NPOS).transpose(2, 1, 0).reshape(FLAT, D)
    h_pre = _conve_apply(subp, irp, w_rel, axr, bxr, g0, b0, g2mat, b96,
                         alpha, beta, fcT, fc_b.reshape(1, D))

    return _score(h_pre, x_pre, axr, bxr, bn2_g.reshape(1, D),
                  bn2_b.reshape(1, D), ent_bias.reshape(1, NUM_ENT))


# speedup vs baseline: 3.9277x; 1.0080x over previous
import functools

import jax
import jax.numpy as jnp
import numpy as np
from jax import lax
from jax.experimental import pallas as pl
from jax.experimental.pallas import tpu as pltpu
from jax.experimental.pallas import tpu_sc as plsc

NUM_ENT = 10000
D = 128
E = 320000
HALF = E // 2
B = 1024
NF = 96
KER = 7
NPOS = 100
FLAT = NF * NPOS

NC, NS = 2, 16
ET = HALF // NS
CH = 80
NCH = ET // CH
WT = 10
RPT = NUM_ENT // WT
ZR = 8

BB = 128
NBB = B // BB
SB = 128
NSB = B // SB

_f32 = jnp.float32


def _edge_aggregate(src_ids, dst_ids, edge_type, edge_norm, init_embed,
                    init_rel):
    mesh = plsc.VectorSubcoreMesh(core_axis_name="c", subcore_axis_name="s")

    NREL = 200

    @functools.partial(
        pl.kernel,
        mesh=mesh,
        out_type=jax.ShapeDtypeStruct((2, NUM_ENT, D), _f32),
        scratch_types=[
            pltpu.VMEM((2, CH), jnp.int32),
            pltpu.VMEM((2, CH), jnp.int32),
            pltpu.VMEM((2, CH), jnp.int32),
            pltpu.VMEM((2, CH), _f32),
            pltpu.VMEM((2, CH, D), _f32),
            pltpu.VMEM((2, CH, D), _f32),
            pltpu.VMEM((ZR, D), _f32),
            pltpu.VMEM_SHARED((NUM_ENT, D), _f32),
            pltpu.SemaphoreType.DMA,
            pltpu.SemaphoreType.DMA,
        ],
    )
    def k(esrc, edst, et, en, emb, rel, out, sidx, didx, tidx, nrm, srows,
          rrows, zbuf, acc, sem1, sem2):
        c = lax.axis_index("c")
        s = lax.axis_index("s")

        def zrow(i, carry):
            for j in range(D // 16):
                zbuf[i, pl.ds(j * 16, 16)] = jnp.zeros((16,), _f32)
            return carry

        lax.fori_loop(0, ZR, zrow, 0)

        @pl.when(s < WT)
        def _():
            for q in range(RPT // ZR):
                pltpu.sync_copy(zbuf, acc.at[pl.ds(s * RPT + q * ZR, ZR)])

        plsc.subcore_barrier()

        base0 = c * HALF + s * ET

        def load_idx(t, b):
            base = base0 + t * CH
            pltpu.sync_copy(esrc.at[pl.ds(base, CH)], sidx.at[b])
            pltpu.sync_copy(edst.at[pl.ds(base, CH)], didx.at[b])
            pltpu.sync_copy(et.at[pl.ds(base, CH)], tidx.at[b])
            pltpu.sync_copy(en.at[pl.ds(base, CH)], nrm.at[b])

        def start_gathers(b, sem):
            pltpu.async_copy(emb.at[sidx.at[b]], srows.at[b], sem)
            pltpu.async_copy(rel.at[tidx.at[b]], rrows.at[b], sem)

        def drain(b, sem):
            pltpu.make_async_copy(emb.at[sidx.at[b]], srows.at[b], sem).wait()
            pltpu.make_async_copy(rel.at[tidx.at[b]], rrows.at[b],
                                  sem).wait()

        def compose_scatter(b):
            def gbody(g, gcarry):
                nvec = nrm[b, pl.ds(g * 16, 16)]
                for i in range(16):
                    nv = lax.gather(
                        nvec, jnp.full((16, 1), i, jnp.int32),
                        lax.GatherDimensionNumbers(
                            offset_dims=(), collapsed_slice_dims=(0,),
                            start_index_map=(0,)),
                        (1,), mode=lax.GatherScatterMode.PROMISE_IN_BOUNDS)
                    e = g * 16 + i
                    for j in range(D // 16):
                        sl = pl.ds(j * 16, 16)
                        srows[b, e, sl] = srows[b, e, sl] * rrows[b, e, sl] * nv
                return gcarry

            lax.fori_loop(0, CH // 16, gbody, 0)
            pltpu.sync_copy(srows.at[b], acc.at[didx.at[b]], add=True)

        load_idx(0, 0)
        start_gathers(0, sem1)

        def body(kk, carry):
            load_idx(2 * kk + 1, 1)
            start_gathers(1, sem2)
            drain(0, sem1)
            compose_scatter(0)
            load_idx(2 * kk + 2, 0)
            start_gathers(0, sem1)
            drain(1, sem2)
            compose_scatter(1)
            return carry

        lax.fori_loop(0, NCH // 2, body, 0)
        drain(0, sem1)
        compose_scatter(0)
        plsc.subcore_barrier()

        @pl.when(s < WT)
        def _():
            pltpu.sync_copy(acc.at[pl.ds(s * RPT, RPT)],
                            out.at[c, pl.ds(s * RPT, RPT)])

    return k(src_ids, dst_ids, edge_type, edge_norm, init_embed, init_rel)


def _gather_rows(x_pre, init_rel, subj, relidx):
    mesh = plsc.VectorSubcoreMesh(core_axis_name="c", subcore_axis_name="s")
    BW = B // (NC * NS)

    @functools.partial(
        pl.kernel,
        mesh=mesh,
        out_type=(jax.ShapeDtypeStruct((B, D), _f32),
                  jax.ShapeDtypeStruct((B, D), _f32)),
        scratch_types=[
            pltpu.VMEM((BW,), jnp.int32),
            pltpu.VMEM((BW,), jnp.int32),
            pltpu.VMEM((BW, D), _f32),
            pltpu.VMEM((BW, D), _f32),
            pltpu.SemaphoreType.DMA,
        ],
    )
    def k(xp, ir, sj, rl, osub, oir, iv1, iv2, r1, r2, sem):
        wid = lax.axis_index("s") * NC + lax.axis_index("c")
        bs = wid * BW
        pltpu.sync_copy(sj.at[pl.ds(bs, BW)], iv1)
        pltpu.sync_copy(rl.at[pl.ds(bs, BW)], iv2)
        cp1 = pltpu.async_copy(xp.at[iv1], r1, sem)
        cp2 = pltpu.async_copy(ir.at[iv2], r2, sem)
        cp1.wait()
        cp2.wait()
        pltpu.sync_copy(r1, osub.at[pl.ds(bs, BW)])
        pltpu.sync_copy(r2, oir.at[pl.ds(bs, BW)])

    return k(x_pre, init_rel, subj, relidx)


def _node_update(acc0, acc1, init_embed, w_in, w_out, w_loop, loop_rel,
                 conv1_bias):
    RA = 2000
    NBA = NUM_ENT // RA

    def body(a0, a1, emb, wi, wo, wl, lr, cb, xout, stat, accsc):
        i = pl.program_id(0)
        z = (jnp.dot(a0[...], wi[...], preferred_element_type=_f32)
             + jnp.dot(a1[...], wo[...], preferred_element_type=_f32)
             + jnp.dot(emb[...] * lr[...], wl[...],
                       preferred_element_type=_f32))
        z = z * (1.0 / 3.0) + cb[...]
        xout[...] = z

        @pl.when(i == 0)
        def _():
            accsc[...] = jnp.zeros_like(accsc)

        accsc[0:1, :] += jnp.sum(z, axis=0, keepdims=True)
        accsc[1:2, :] += jnp.sum(z * z, axis=0, keepdims=True)
        stat[...] = accsc[...]

    full = lambda shape: pl.BlockSpec(shape, lambda i: (0,) * len(shape))
    return pl.pallas_call(
        body,
        grid=(NBA,),
        in_specs=[
            pl.BlockSpec((RA, D), lambda i: (i, 0)),
            pl.BlockSpec((RA, D), lambda i: (i, 0)),
            pl.BlockSpec((RA, D), lambda i: (i, 0)),
            full((D, D)), full((D, D)), full((D, D)),
            full((1, D)), full((1, D)),
        ],
        out_specs=[pl.BlockSpec((RA, D), lambda i: (i, 0)), full((2, D))],
        out_shape=[jax.ShapeDtypeStruct((NUM_ENT, D), _f32),
                   jax.ShapeDtypeStruct((2, D), _f32)],
        scratch_shapes=[pltpu.VMEM((2, D), _f32)],
    )(acc0, acc1, init_embed, w_in, w_out, w_loop, loop_rel, conv1_bias)


def _tanh_rows(x_pre, axr, bxr):
    RT = 2000

    def body(x, ax, bx, o):
        o[...] = jnp.tanh(x[...] * ax[...] + bx[...])

    full = lambda shape: pl.BlockSpec(shape, lambda i: (0,) * len(shape))
    return pl.pallas_call(
        body,
        grid=(NUM_ENT // RT,),
        in_specs=[pl.BlockSpec((RT, D), lambda i: (i, 0)),
                  full((1, D)), full((1, D))],
        out_specs=pl.BlockSpec((RT, D), lambda i: (i, 0)),
        out_shape=jax.ShapeDtypeStruct((NUM_ENT, D), _f32),
    )(x_pre, axr, bxr)


def _rel_transform(init_rel, w_rel):
    def body(ir, w, o):
        o[...] = jnp.dot(ir[...], w[...], preferred_element_type=_f32)

    nrel = init_rel.shape[0]
    full = lambda shape: pl.BlockSpec(shape, lambda: (0,) * len(shape))
    return pl.pallas_call(
        body,
        in_specs=[full((nrel, D)), full((D, D))],
        out_specs=full((nrel, D)),
        out_shape=jax.ShapeDtypeStruct((nrel, D), _f32),
    )(init_rel, w_rel)


def _conve_stats(subp, irp, g0, b0, g2mat, b96):
    def body(subf, irf, subb, irb, gg0, bb0, g2m, bb96, convout, stat,
             accsc, ssc):
        i = pl.program_id(0)

        @pl.when(i == 0)
        def _():
            n0 = 2.0 * B * D
            s0 = jnp.sum(subf[...]) + jnp.sum(irf[...])
            ss0 = (jnp.sum(subf[...] * subf[...])
                   + jnp.sum(irf[...] * irf[...]))
            m0 = s0 / n0
            v0 = ss0 / n0 - m0 * m0
            sc0 = gg0[0, 0] * lax.rsqrt(v0 + 1e-5)
            ssc[0] = sc0
            ssc[1] = bb0[0, 0] - m0 * sc0
            accsc[...] = jnp.zeros_like(accsc)

        sc0 = ssc[0]
        sh0 = ssc[1]
        img = jnp.concatenate([subb[...] * sc0 + sh0,
                               irb[...] * sc0 + sh0], axis=1)
        conv = jnp.dot(img, g2m[...], preferred_element_type=_f32) + bb96[...]
        convout[...] = conv
        accsc[0:1, :] += jnp.sum(conv, axis=0, keepdims=True)
        accsc[1:2, :] += jnp.sum(conv * conv, axis=0, keepdims=True)
        stat[...] = accsc[...]

    full = lambda shape: pl.BlockSpec(shape, lambda i: (0,) * len(shape))
    return pl.pallas_call(
        body,
        grid=(NBB,),
        in_specs=[
            full((B, D)), full((B, D)),
            pl.BlockSpec((BB, D), lambda i: (i, 0)),
            pl.BlockSpec((BB, D), lambda i: (i, 0)),
            full((1, 1)), full((1, 1)),
            full((2 * D, FLAT)), full((1, FLAT)),
        ],
        out_specs=[pl.BlockSpec((BB, FLAT), lambda i: (i, 0)),
                   full((2, FLAT))],
        out_shape=[jax.ShapeDtypeStruct((B, FLAT), _f32),
                   jax.ShapeDtypeStruct((2, FLAT), _f32)],
        scratch_shapes=[pltpu.VMEM((2, FLAT), _f32),
                        pltpu.SMEM((2,), _f32)],
    )(subp, irp, subp, irp, g0, b0, g2mat, b96)


def _conve_apply(conv, alpha, beta, fcT, fcb):
    def body(cv, al, be, fw, fb, hout):
        y = jnp.maximum(cv[...] * al[...] + be[...], 0.0)
        hout[...] = jnp.dot(y, fw[...], preferred_element_type=_f32) + fb[...]

    full = lambda shape: pl.BlockSpec(shape, lambda i: (0,) * len(shape))
    return pl.pallas_call(
        body,
        grid=(NBB,),
        in_specs=[
            pl.BlockSpec((BB, FLAT), lambda i: (i, 0)),
            full((1, FLAT)), full((1, FLAT)),
            full((FLAT, D)), full((1, D)),
        ],
        out_specs=pl.BlockSpec((BB, D), lambda i: (i, 0)),
        out_shape=jax.ShapeDtypeStruct((B, D), _f32),
    )(conv, alpha, beta, fcT, fcb)


def _score(h_pre, x_tanh, g2r, b2r, ent_bias):
    def body(hf, hb_ref, xt, g2, b2, eb, score):
        hp = hf[...]
        m = jnp.mean(hp, axis=0, keepdims=True)
        v = jnp.mean(hp * hp, axis=0, keepdims=True) - m * m
        sc2 = lax.rsqrt(v + 1e-5) * g2[...]
        hb = jnp.maximum((hb_ref[...] - m) * sc2 + b2[...], 0.0)
        sc = lax.dot_general(hb, xt[...], (((1,), (1,)), ((), ())),
                             preferred_element_type=_f32)
        score[...] = 1.0 / (1.0 + jnp.exp(-(sc + eb[...])))

    full = lambda shape: pl.BlockSpec(shape, lambda i: (0,) * len(shape))
    return pl.pallas_call(
        body,
        grid=(NSB,),
        in_specs=[
            full((B, D)),
            pl.BlockSpec((SB, D), lambda i: (i, 0)),
            full((NUM_ENT, D)),
            full((1, D)), full((1, D)),
            full((1, NUM_ENT)),
        ],
        out_specs=pl.BlockSpec((SB, NUM_ENT), lambda i: (i, 0)),
        out_shape=jax.ShapeDtypeStruct((B, NUM_ENT), _f32),
    )(h_pre, h_pre, x_tanh, g2r, b2r, ent_bias)


def _toeplitz_selector():
    a = np.zeros((2 * D * NPOS, KER * KER), np.float32)
    r = np.arange(2 * D)
    p = np.arange(NPOS)
    R, P = np.meshgrid(r, p, indexing="ij")
    ky = R // 16 - P // 10
    kx = R % 16 - P % 10
    ok = (ky >= 0) & (ky < KER) & (kx >= 0) & (kx < KER)
    t = np.where(ok, ky * KER + kx, 0)
    a[np.arange(2 * D * NPOS)[ok.ravel()], t.ravel()[ok.ravel()]] = 1.0
    return a


_A_SEL = _toeplitz_selector()
GR = 6400
NGR = (2 * D * NPOS) // GR


def _build_toeplitz(conv2d_w):
    w_flat = conv2d_w.reshape(NF, KER * KER)

    def body(a, w, o):
        o[...] = lax.dot_general(a[...], w[...], (((1,), (1,)), ((), ())),
                                 preferred_element_type=_f32)

    g = pl.pallas_call(
        body,
        grid=(NGR,),
        in_specs=[pl.BlockSpec((GR, KER * KER), lambda i: (i, 0)),
                  pl.BlockSpec((NF, KER * KER), lambda i: (0, 0))],
        out_specs=pl.BlockSpec((GR, NF), lambda i: (i, 0)),
        out_shape=jax.ShapeDtypeStruct((2 * D * NPOS, NF), _f32),
    )(jnp.asarray(_A_SEL), w_flat)
    return g.reshape(2 * D, FLAT)


def kernel(edge_index, edge_type, edge_norm, subj, rel, init_embed, init_rel,
           w_loop, w_in, w_out, w_rel, loop_rel, conv1_bias, bn_c1_g, bn_c1_b,
           ent_bias, bn0_g, bn0_b, conv2d_w, conv2d_b, bn1_g, bn1_b, fc_w,
           fc_b, bn2_g, bn2_b):
    r_all = _rel_transform(init_rel, w_rel)
    g2mat = _build_toeplitz(conv2d_w)

    acc = _edge_aggregate(edge_index[0], edge_index[1], edge_type, edge_norm,
                          init_embed, init_rel)

    x_pre, xstat = _node_update(acc[0], acc[1], init_embed, w_in, w_out,
                                w_loop, loop_rel.reshape(1, D),
                                conv1_bias.reshape(1, D))
    mean_x = xstat[0] / NUM_ENT
    var_x = xstat[1] / NUM_ENT - mean_x * mean_x
    ax = bn_c1_g * lax.rsqrt(var_x + 1e-5)
    bx = bn_c1_b - mean_x * ax
    axr, bxr = ax.reshape(1, D), bx.reshape(1, D)

    x_tanh = _tanh_rows(x_pre, axr, bxr)

    subp, irp = _gather_rows(x_tanh, r_all, subj, rel)

    b96 = jnp.tile(conv2d_b, NPOS).reshape(1, FLAT)
    g0 = bn0_g.reshape(1, 1)
    b0 = bn0_b.reshape(1, 1)

    conv, cstat = _conve_stats(subp, irp, g0, b0, g2mat, b96)
    n1 = float(B * NPOS)
    sums = cstat[0].reshape(NPOS, NF).sum(axis=0)
    sumsq = cstat[1].reshape(NPOS, NF).sum(axis=0)
    mf = sums / n1
    vf = sumsq / n1 - mf * mf
    af = bn1_g * lax.rsqrt(vf + 1e-5)
    bf = bn1_b - mf * af
    alpha = jnp.tile(af, NPOS).reshape(1, FLAT)
    beta = jnp.tile(bf, NPOS).reshape(1, FLAT)

    fcT = fc_w.reshape(D, NF, NPOS).transpose(2, 1, 0).reshape(FLAT, D)
    h_pre = _conve_apply(conv, alpha, beta, fcT, fc_b.reshape(1, D))

    return _score(h_pre, x_tanh, bn2_g.reshape(1, D),
                  bn2_b.reshape(1, D), ent_bias.reshape(1, NUM_ENT))


# speedup vs baseline: 4.1085x; 1.0460x over previous
import functools

import jax
import jax.numpy as jnp
import numpy as np
from jax import lax
from jax.experimental import pallas as pl
from jax.experimental.pallas import tpu as pltpu
from jax.experimental.pallas import tpu_sc as plsc

NUM_ENT = 10000
D = 128
E = 320000
HALF = E // 2
B = 1024
NF = 96
KER = 7
NPOS = 100
FLAT = NF * NPOS

NC, NS = 2, 16
ET = HALF // NS
CH = 80
NCH = ET // CH
WT = 10
RPT = NUM_ENT // WT
ZR = 8

BB = 128
NBB = B // BB
SB = 128
NSB = B // SB

_f32 = jnp.float32


def _edge_aggregate(src_ids, dst_ids, edge_type, edge_norm, init_embed,
                    init_rel):
    mesh = plsc.VectorSubcoreMesh(core_axis_name="c", subcore_axis_name="s")

    NREL = 200

    NB = 2

    @functools.partial(
        pl.kernel,
        mesh=mesh,
        out_type=jax.ShapeDtypeStruct((2, NUM_ENT, D), _f32),
        scratch_types=[
            pltpu.VMEM((NB, CH), jnp.int32),
            pltpu.VMEM((NB, CH), jnp.int32),
            pltpu.VMEM((NB, CH), jnp.int32),
            pltpu.VMEM((NB, CH), _f32),
            pltpu.VMEM((NB, CH, D), _f32),
            pltpu.VMEM((NB, CH, D), _f32),
            pltpu.VMEM((ZR, D), _f32),
            pltpu.VMEM_SHARED((NUM_ENT, D), _f32),
        ] + [pltpu.SemaphoreType.DMA] * (2 * NB),
    )
    def k(esrc, edst, et, en, emb, rel, out, sidx, didx, tidx, nrm, srows,
          rrows, zbuf, acc, *sems):
        gsem = sems[:NB]
        ssem = sems[NB:]
        c = lax.axis_index("c")
        s = lax.axis_index("s")

        def zrow(i, carry):
            for j in range(D // 16):
                zbuf[i, pl.ds(j * 16, 16)] = jnp.zeros((16,), _f32)
            return carry

        lax.fori_loop(0, ZR, zrow, 0)

        @pl.when(s < WT)
        def _():
            for q in range(RPT // ZR):
                pltpu.sync_copy(zbuf, acc.at[pl.ds(s * RPT + q * ZR, ZR)])

        plsc.subcore_barrier()

        base0 = c * HALF + s * ET

        def load_idx(t, b):
            base = base0 + t * CH
            pltpu.sync_copy(esrc.at[pl.ds(base, CH)], sidx.at[b])
            pltpu.sync_copy(edst.at[pl.ds(base, CH)], didx.at[b])
            pltpu.sync_copy(et.at[pl.ds(base, CH)], tidx.at[b])
            pltpu.sync_copy(en.at[pl.ds(base, CH)], nrm.at[b])

        def start_gathers(b):
            pltpu.async_copy(emb.at[sidx.at[b]], srows.at[b], gsem[b])
            pltpu.async_copy(rel.at[tidx.at[b]], rrows.at[b], gsem[b])

        def drain(b):
            pltpu.make_async_copy(emb.at[sidx.at[b]], srows.at[b],
                                  gsem[b]).wait()
            pltpu.make_async_copy(rel.at[tidx.at[b]], rrows.at[b],
                                  gsem[b]).wait()

        def compose(b):
            def gbody(g, gcarry):
                nvec = nrm[b, pl.ds(g * 16, 16)]
                for i in range(16):
                    nv = lax.gather(
                        nvec, jnp.full((16, 1), i, jnp.int32),
                        lax.GatherDimensionNumbers(
                            offset_dims=(), collapsed_slice_dims=(0,),
                            start_index_map=(0,)),
                        (1,), mode=lax.GatherScatterMode.PROMISE_IN_BOUNDS)
                    e = g * 16 + i
                    for j in range(D // 16):
                        sl = pl.ds(j * 16, 16)
                        srows[b, e, sl] = srows[b, e, sl] * rrows[b, e, sl] * nv
                return gcarry

            lax.fori_loop(0, CH // 16, gbody, 0)

        def start_scatter(b):
            pltpu.async_copy(srows.at[b], acc.at[didx.at[b]], ssem[b],
                             add=True)

        def wait_scatter(b):
            pltpu.make_async_copy(srows.at[b], acc.at[didx.at[b]],
                                  ssem[b]).wait()

        load_idx(NCH - 1, 0)
        start_gathers(0)
        drain(0)
        compose(0)
        start_scatter(0)
        wait_scatter(0)

        NCHE = NCH - 1
        load_idx(0, 0)
        start_gathers(0)
        load_idx(1, 1)
        start_gathers(1)

        def body(kk, carry):
            t0 = 2 * kk
            drain(0)
            compose(0)
            start_scatter(0)
            drain(1)
            compose(1)
            start_scatter(1)
            wait_scatter(0)
            load_idx(lax.rem(t0 + 2, NCHE), 0)
            start_gathers(0)
            wait_scatter(1)
            load_idx(lax.rem(t0 + 3, NCHE), 1)
            start_gathers(1)
            return carry

        lax.fori_loop(0, NCHE // 2, body, 0)
        drain(0)
        drain(1)

        plsc.subcore_barrier()

        @pl.when(s < WT)
        def _():
            pltpu.sync_copy(acc.at[pl.ds(s * RPT, RPT)],
                            out.at[c, pl.ds(s * RPT, RPT)])

    return k(src_ids, dst_ids, edge_type, edge_norm, init_embed, init_rel)


def _gather_rows(x_pre, init_rel, subj, relidx):
    mesh = plsc.VectorSubcoreMesh(core_axis_name="c", subcore_axis_name="s")
    BW = B // (NC * NS)

    @functools.partial(
        pl.kernel,
        mesh=mesh,
        out_type=(jax.ShapeDtypeStruct((B, D), _f32),
                  jax.ShapeDtypeStruct((B, D), _f32)),
        scratch_types=[
            pltpu.VMEM((BW,), jnp.int32),
            pltpu.VMEM((BW,), jnp.int32),
            pltpu.VMEM((BW, D), _f32),
            pltpu.VMEM((BW, D), _f32),
            pltpu.SemaphoreType.DMA,
        ],
    )
    def k(xp, ir, sj, rl, osub, oir, iv1, iv2, r1, r2, sem):
        wid = lax.axis_index("s") * NC + lax.axis_index("c")
        bs = wid * BW
        pltpu.sync_copy(sj.at[pl.ds(bs, BW)], iv1)
        pltpu.sync_copy(rl.at[pl.ds(bs, BW)], iv2)
        cp1 = pltpu.async_copy(xp.at[iv1], r1, sem)
        cp2 = pltpu.async_copy(ir.at[iv2], r2, sem)
        cp1.wait()
        cp2.wait()
        pltpu.sync_copy(r1, osub.at[pl.ds(bs, BW)])
        pltpu.sync_copy(r2, oir.at[pl.ds(bs, BW)])

    return k(x_pre, init_rel, subj, relidx)


def _node_update(acc0, acc1, init_embed, w_in, w_out, w_loop, loop_rel,
                 conv1_bias):
    RA = 2000
    NBA = NUM_ENT // RA

    def body(a0, a1, emb, wi, wo, wl, lr, cb, xout, stat, accsc):
        i = pl.program_id(0)
        z = (jnp.dot(a0[...], wi[...], preferred_element_type=_f32)
             + jnp.dot(a1[...], wo[...], preferred_element_type=_f32)
             + jnp.dot(emb[...] * lr[...], wl[...],
                       preferred_element_type=_f32))
        z = z * (1.0 / 3.0) + cb[...]
        xout[...] = z

        @pl.when(i == 0)
        def _():
            accsc[...] = jnp.zeros_like(accsc)

        accsc[0:1, :] += jnp.sum(z, axis=0, keepdims=True)
        accsc[1:2, :] += jnp.sum(z * z, axis=0, keepdims=True)
        stat[...] = accsc[...]

    full = lambda shape: pl.BlockSpec(shape, lambda i: (0,) * len(shape))
    return pl.pallas_call(
        body,
        grid=(NBA,),
        in_specs=[
            pl.BlockSpec((RA, D), lambda i: (i, 0)),
            pl.BlockSpec((RA, D), lambda i: (i, 0)),
            pl.BlockSpec((RA, D), lambda i: (i, 0)),
            full((D, D)), full((D, D)), full((D, D)),
            full((1, D)), full((1, D)),
        ],
        out_specs=[pl.BlockSpec((RA, D), lambda i: (i, 0)), full((2, D))],
        out_shape=[jax.ShapeDtypeStruct((NUM_ENT, D), _f32),
                   jax.ShapeDtypeStruct((2, D), _f32)],
        scratch_shapes=[pltpu.VMEM((2, D), _f32)],
    )(acc0, acc1, init_embed, w_in, w_out, w_loop, loop_rel, conv1_bias)


def _tanh_rows(x_pre, axr, bxr):
    RT = 2000

    def body(x, ax, bx, o):
        o[...] = jnp.tanh(x[...] * ax[...] + bx[...])

    full = lambda shape: pl.BlockSpec(shape, lambda i: (0,) * len(shape))
    return pl.pallas_call(
        body,
        grid=(NUM_ENT // RT,),
        in_specs=[pl.BlockSpec((RT, D), lambda i: (i, 0)),
                  full((1, D)), full((1, D))],
        out_specs=pl.BlockSpec((RT, D), lambda i: (i, 0)),
        out_shape=jax.ShapeDtypeStruct((NUM_ENT, D), _f32),
    )(x_pre, axr, bxr)


def _rel_transform(init_rel, w_rel):
    def body(ir, w, o):
        o[...] = jnp.dot(ir[...], w[...], preferred_element_type=_f32)

    nrel = init_rel.shape[0]
    full = lambda shape: pl.BlockSpec(shape, lambda: (0,) * len(shape))
    return pl.pallas_call(
        body,
        in_specs=[full((nrel, D)), full((D, D))],
        out_specs=full((nrel, D)),
        out_shape=jax.ShapeDtypeStruct((nrel, D), _f32),
    )(init_rel, w_rel)


def _conve_stats(subp, irp, g0, b0, g2mat, b96):
    def body(subf, irf, subb, irb, gg0, bb0, g2m, bb96, convout, stat,
             accsc, ssc):
        i = pl.program_id(0)

        @pl.when(i == 0)
        def _():
            n0 = 2.0 * B * D
            s0 = jnp.sum(subf[...]) + jnp.sum(irf[...])
            ss0 = (jnp.sum(subf[...] * subf[...])
                   + jnp.sum(irf[...] * irf[...]))
            m0 = s0 / n0
            v0 = ss0 / n0 - m0 * m0
            sc0 = gg0[0, 0] * lax.rsqrt(v0 + 1e-5)
            ssc[0] = sc0
            ssc[1] = bb0[0, 0] - m0 * sc0
            accsc[...] = jnp.zeros_like(accsc)

        sc0 = ssc[0]
        sh0 = ssc[1]
        img = jnp.concatenate([subb[...] * sc0 + sh0,
                               irb[...] * sc0 + sh0], axis=1)
        conv = jnp.dot(img, g2m[...], preferred_element_type=_f32) + bb96[...]
        convout[...] = conv
        accsc[0:1, :] += jnp.sum(conv, axis=0, keepdims=True)
        accsc[1:2, :] += jnp.sum(conv * conv, axis=0, keepdims=True)
        stat[...] = accsc[...]

    full = lambda shape: pl.BlockSpec(shape, lambda i: (0,) * len(shape))
    return pl.pallas_call(
        body,
        grid=(NBB,),
        in_specs=[
            full((B, D)), full((B, D)),
            pl.BlockSpec((BB, D), lambda i: (i, 0)),
            pl.BlockSpec((BB, D), lambda i: (i, 0)),
            full((1, 1)), full((1, 1)),
            full((2 * D, FLAT)), full((1, FLAT)),
        ],
        out_specs=[pl.BlockSpec((BB, FLAT), lambda i: (i, 0)),
                   full((2, FLAT))],
        out_shape=[jax.ShapeDtypeStruct((B, FLAT), _f32),
                   jax.ShapeDtypeStruct((2, FLAT), _f32)],
        scratch_shapes=[pltpu.VMEM((2, FLAT), _f32),
                        pltpu.SMEM((2,), _f32)],
    )(subp, irp, subp, irp, g0, b0, g2mat, b96)


def _conve_apply(conv, alpha, beta, fcT, fcb):
    def body(cv, al, be, fw, fb, hout):
        y = jnp.maximum(cv[...] * al[...] + be[...], 0.0)
        hout[...] = jnp.dot(y, fw[...], preferred_element_type=_f32) + fb[...]

    full = lambda shape: pl.BlockSpec(shape, lambda i: (0,) * len(shape))
    return pl.pallas_call(
        body,
        grid=(NBB,),
        in_specs=[
            pl.BlockSpec((BB, FLAT), lambda i: (i, 0)),
            full((1, FLAT)), full((1, FLAT)),
            full((FLAT, D)), full((1, D)),
        ],
        out_specs=pl.BlockSpec((BB, D), lambda i: (i, 0)),
        out_shape=jax.ShapeDtypeStruct((B, D), _f32),
    )(conv, alpha, beta, fcT, fcb)


def _score(h_pre, x_tanh, g2r, b2r, ent_bias):
    def body(hf, hb_ref, xt, g2, b2, eb, score):
        hp = hf[...]
        m = jnp.mean(hp, axis=0, keepdims=True)
        v = jnp.mean(hp * hp, axis=0, keepdims=True) - m * m
        sc2 = lax.rsqrt(v + 1e-5) * g2[...]
        hb = jnp.maximum((hb_ref[...] - m) * sc2 + b2[...], 0.0)
        sc = lax.dot_general(hb, xt[...], (((1,), (1,)), ((), ())),
                             preferred_element_type=_f32)
        score[...] = 1.0 / (1.0 + jnp.exp(-(sc + eb[...])))

    full = lambda shape: pl.BlockSpec(shape, lambda i: (0,) * len(shape))
    return pl.pallas_call(
        body,
        grid=(NSB,),
        in_specs=[
            full((B, D)),
            pl.BlockSpec((SB, D), lambda i: (i, 0)),
            full((NUM_ENT, D)),
            full((1, D)), full((1, D)),
            full((1, NUM_ENT)),
        ],
        out_specs=pl.BlockSpec((SB, NUM_ENT), lambda i: (i, 0)),
        out_shape=jax.ShapeDtypeStruct((B, NUM_ENT), _f32),
    )(h_pre, h_pre, x_tanh, g2r, b2r, ent_bias)


def _toeplitz_selector():
    a = np.zeros((2 * D * NPOS, KER * KER), np.float32)
    r = np.arange(2 * D)
    p = np.arange(NPOS)
    R, P = np.meshgrid(r, p, indexing="ij")
    ky = R // 16 - P // 10
    kx = R % 16 - P % 10
    ok = (ky >= 0) & (ky < KER) & (kx >= 0) & (kx < KER)
    t = np.where(ok, ky * KER + kx, 0)
    a[np.arange(2 * D * NPOS)[ok.ravel()], t.ravel()[ok.ravel()]] = 1.0
    return a


_A_SEL = _toeplitz_selector()
GR = 6400
NGR = (2 * D * NPOS) // GR


def _build_toeplitz(conv2d_w):
    w_flat = conv2d_w.reshape(NF, KER * KER)

    def body(a, w, o):
        o[...] = lax.dot_general(a[...], w[...], (((1,), (1,)), ((), ())),
                                 preferred_element_type=_f32)

    g = pl.pallas_call(
        body,
        grid=(NGR,),
        in_specs=[pl.BlockSpec((GR, KER * KER), lambda i: (i, 0)),
                  pl.BlockSpec((NF, KER * KER), lambda i: (0, 0))],
        out_specs=pl.BlockSpec((GR, NF), lambda i: (i, 0)),
        out_shape=jax.ShapeDtypeStruct((2 * D * NPOS, NF), _f32),
    )(jnp.asarray(_A_SEL), w_flat)
    return g.reshape(2 * D, FLAT)


def kernel(edge_index, edge_type, edge_norm, subj, rel, init_embed, init_rel,
           w_loop, w_in, w_out, w_rel, loop_rel, conv1_bias, bn_c1_g, bn_c1_b,
           ent_bias, bn0_g, bn0_b, conv2d_w, conv2d_b, bn1_g, bn1_b, fc_w,
           fc_b, bn2_g, bn2_b):
    r_all = _rel_transform(init_rel, w_rel)
    g2mat = _build_toeplitz(conv2d_w)

    acc = _edge_aggregate(edge_index[0], edge_index[1], edge_type, edge_norm,
                          init_embed, init_rel)

    x_pre, xstat = _node_update(acc[0], acc[1], init_embed, w_in, w_out,
                                w_loop, loop_rel.reshape(1, D),
                                conv1_bias.reshape(1, D))
    mean_x = xstat[0] / NUM_ENT
    var_x = xstat[1] / NUM_ENT - mean_x * mean_x
    ax = bn_c1_g * lax.rsqrt(var_x + 1e-5)
    bx = bn_c1_b - mean_x * ax
    axr, bxr = ax.reshape(1, D), bx.reshape(1, D)

    x_tanh = _tanh_rows(x_pre, axr, bxr)

    subp, irp = _gather_rows(x_tanh, r_all, subj, rel)

    b96 = jnp.tile(conv2d_b, NPOS).reshape(1, FLAT)
    g0 = bn0_g.reshape(1, 1)
    b0 = bn0_b.reshape(1, 1)

    conv, cstat = _conve_stats(subp, irp, g0, b0, g2mat, b96)
    n1 = float(B * NPOS)
    sums = cstat[0].reshape(NPOS, NF).sum(axis=0)
    sumsq = cstat[1].reshape(NPOS, NF).sum(axis=0)
    mf = sums / n1
    vf = sumsq / n1 - mf * mf
    af = bn1_g * lax.rsqrt(vf + 1e-5)
    bf = bn1_b - mf * af
    alpha = jnp.tile(af, NPOS).reshape(1, FLAT)
    beta = jnp.tile(bf, NPOS).reshape(1, FLAT)

    fcT = fc_w.reshape(D, NF, NPOS).transpose(2, 1, 0).reshape(FLAT, D)
    h_pre = _conve_apply(conv, alpha, beta, fcT, fc_b.reshape(1, D))

    return _score(h_pre, x_tanh, bn2_g.reshape(1, D),
                  bn2_b.reshape(1, D), ent_bias.reshape(1, NUM_ENT))
